# baseline (speedup 1.0000x reference)
import jax
import jax.numpy as jnp
from jax.experimental import pallas as pl
from jax.experimental.pallas import tpu as pltpu

_N_OUT = 100
_NC_PER_TASK = 10
_NEG_FILL = -100000000000.0
_BN = 256


def _fused_linear_mask_kernel(t_ref, xt_ref, wt_ref, b_ref, o_ref):
    off1 = t_ref[0] * _NC_PER_TASK
    off2 = off1 + _NC_PER_TASK
    xb = xt_ref[...].astype(jnp.bfloat16)
    wb = wt_ref[...].astype(jnp.bfloat16)
    acc = jnp.dot(wb, xb, preferred_element_type=jnp.float32)
    rows = jax.lax.broadcasted_iota(jnp.int32, (_N_OUT, 1), 0)
    keep = (rows >= off1) & (rows < off2)
    o_ref[...] = jnp.where(keep, acc + b_ref[...], _NEG_FILL)


def kernel(x, W, b, t):
    B = x.shape[0]
    K = x.size // B
    xT = x.transpose(1, 2, 3, 0).reshape(K, B)
    WT = W.T
    t_arr = jnp.atleast_1d(jnp.asarray(t, jnp.int32))
    bT = b.reshape(_N_OUT, 1)
    grid = (B // _BN,)
    outT = pl.pallas_call(
        _fused_linear_mask_kernel,
        grid_spec=pltpu.PrefetchScalarGridSpec(
            num_scalar_prefetch=1,
            grid=grid,
            in_specs=[
                pl.BlockSpec((K, _BN), lambda i, t_s: (0, i)),
                pl.BlockSpec((_N_OUT, K), lambda i, t_s: (0, 0)),
                pl.BlockSpec((_N_OUT, 1), lambda i, t_s: (0, 0)),
            ],
            out_specs=pl.BlockSpec((_N_OUT, _BN), lambda i, t_s: (0, i)),
        ),
        out_shape=jax.ShapeDtypeStruct((_N_OUT, B), jnp.float32),
        compiler_params=pltpu.CompilerParams(
            dimension_semantics=("arbitrary",),
        ),
    )(t_arr, xT, WT, bT)
    return outT.T


# speedup vs baseline: 1.2413x; 1.2413x over previous
import jax
import jax.numpy as jnp
from jax.experimental import pallas as pl
from jax.experimental.pallas import tpu as pltpu

_N_OUT = 100
_NC_PER_TASK = 10
_NEG_FILL = -100000000000.0
_BN = 1024
_NSPLIT = 2


def _fused_linear_mask_kernel(*refs):
    t_ref = refs[0]
    x_refs = refs[1:1 + _NSPLIT]
    wt_ref, b_ref, o_ref = refs[1 + _NSPLIT:]
    off1 = t_ref[0] * _NC_PER_TASK
    off2 = off1 + _NC_PER_TASK
    kq = x_refs[0].shape[0]
    acc = jnp.zeros((_N_OUT, x_refs[0].shape[1]), jnp.float32)
    for j, xr in enumerate(x_refs):
        xb = xr[...].astype(jnp.bfloat16)
        wb = wt_ref[:, j * kq:(j + 1) * kq].astype(jnp.bfloat16)
        acc = acc + jnp.dot(wb, xb, preferred_element_type=jnp.float32)
    rows = jax.lax.broadcasted_iota(jnp.int32, (_N_OUT, 1), 0)
    keep = (rows >= off1) & (rows < off2)
    o_ref[...] = jnp.where(keep, acc + b_ref[...], _NEG_FILL)


def kernel(x, W, b, t):
    B = x.shape[0]
    K = x.size // B
    kq = K // _NSPLIT
    xT = x.transpose(1, 2, 3, 0).reshape(K, B)
    WT = W.T
    t_arr = jnp.atleast_1d(jnp.asarray(t, jnp.int32))
    bT = b.reshape(_N_OUT, 1)
    grid = (B // _BN,)

    def make_xspec(j):
        return pl.BlockSpec((kq, _BN), lambda i, t_s, j=j: (j, i))

    outT = pl.pallas_call(
        _fused_linear_mask_kernel,
        grid_spec=pltpu.PrefetchScalarGridSpec(
            num_scalar_prefetch=1,
            grid=grid,
            in_specs=[make_xspec(j) for j in range(_NSPLIT)] + [
                pl.BlockSpec((_N_OUT, K), lambda i, t_s: (0, 0)),
                pl.BlockSpec((_N_OUT, 1), lambda i, t_s: (0, 0)),
            ],
            out_specs=pl.BlockSpec((_N_OUT, _BN), lambda i, t_s: (0, i)),
        ),
        out_shape=jax.ShapeDtypeStruct((_N_OUT, B), jnp.float32),
        compiler_params=pltpu.CompilerParams(
            dimension_semantics=("arbitrary",),
        ),
    )(t_arr, *([xT] * _NSPLIT), WT, bT)
    return outT.T
